# baseline (speedup 1.0000x reference)
import jax
import jax.numpy as jnp
import numpy as np
from jax.experimental import pallas as pl
from jax.experimental.pallas import tpu as pltpu

_N = 16384
_D = 768
_T = 2048
_B = 4
_DECAY = 0.9
_ROUNDS = 10
_THRESH = 0.5

_NB = 16
_BN = _N // _NB
_NH = _NB // 2
_TB = 16
_BT = _T // _TB
_SCALE = np.float32(1.0 / np.sqrt(np.float32(_D)))


def _body(qs_lo, qs_hi, keys_lo, keys_hi, val_lo, val_hi,
          pri_ref, ages_ref, vm_ref, wq_ref, bq_ref, wc_ref, bc_ref,
          out_ref, qvec, s_scr, w_scr, acc):
    p = pl.program_id(0)
    j = pl.program_id(1)
    f32 = jnp.float32

    @pl.when(jnp.logical_and(p == 0, j == 0))
    def _init():
        qvec[...] = jnp.zeros_like(qvec)
        acc[...] = jnp.zeros_like(acc)

    @pl.when(p == 0)
    def _pool():
        qvec[...] += (jnp.sum(qs_lo[...], axis=1)
                      + jnp.sum(qs_hi[...], axis=1))

    @pl.when(jnp.logical_and(p == 1, j == 0))
    def _project_q():
        q = qvec[...] * (1.0 / _T)
        qvec[...] = jax.lax.dot_general(
            q, wq_ref[...], (((1,), (1,)), ((), ())),
            preferred_element_type=f32) + bq_ref[...]

    @pl.when(p == 1)
    def _scores():
        qb = qvec[...].astype(jnp.bfloat16)
        s_scr[j] = jax.lax.dot_general(
            qb, keys_lo[...].astype(jnp.bfloat16), (((1,), (1,)), ((), ())),
            preferred_element_type=f32) * _SCALE
        s_scr[j + _NH] = jax.lax.dot_general(
            qb, keys_hi[...].astype(jnp.bfloat16), (((1,), (1,)), ((), ())),
            preferred_element_type=f32) * _SCALE

    @pl.when(jnp.logical_and(p == 2, j == 0))
    def _rounds():
        s = s_scr[...]
        log_decay = np.float32(np.log(_DECAY))
        eff0 = pri_ref[...] * jnp.exp(ages_ref[...] * log_decay)
        vm = vm_ref[...]
        wsum = jnp.zeros_like(s)
        for r in range(_ROUNDS):
            eff = eff0 * np.float32(_DECAY ** r)
            logits = s + jnp.log(eff + 1e-8)
            m = jnp.max(logits, axis=(0, 2), keepdims=True)
            pex = jnp.exp(logits - m)
            den = jnp.sum(pex, axis=(0, 2), keepdims=True)
            active = jax.nn.sigmoid((eff - _THRESH) * 10.0) * vm
            num = pex * active
            wsum += num / (jnp.sum(num, axis=(0, 2), keepdims=True)
                           + 1e-8 * den)
        w_scr[...] = wsum

    @pl.when(p == 2)
    def _retrieve():
        acc[...] += (jax.lax.dot_general(
            w_scr[j].astype(jnp.bfloat16),
            val_lo[...].astype(jnp.bfloat16), (((1,), (0,)), ((), ())),
            preferred_element_type=f32)
            + jax.lax.dot_general(
                w_scr[j + _NH].astype(jnp.bfloat16),
                val_hi[...].astype(jnp.bfloat16), (((1,), (0,)), ((), ())),
                preferred_element_type=f32))

    @pl.when(jnp.logical_and(p == 2, j == _NH - 1))
    def _project_out():
        out_ref[...] = jax.lax.dot_general(
            acc[...], wc_ref[...], (((1,), (1,)), ((), ())),
            preferred_element_type=f32) * (1.0 / _ROUNDS) + bc_ref[...]


@jax.jit
def kernel(query_states, keys, values, priorities, Wq, bq, Wc, bc, ages,
           valid_mask):
    B, T, D = query_states.shape

    pri = priorities.reshape(_NB, 1, _BN)
    ages_f = ages.astype(jnp.float32).reshape(_NB, 1, _BN)
    vm = valid_mask.astype(jnp.float32).reshape(_NB, 1, _BN)
    bq2 = bq.reshape(1, D)
    bc2 = bc.reshape(1, D)

    th = _TB // 2
    qs_lo_map = lambda p, j: (0, jnp.where(p == 0, j, th - 1), 0)
    qs_hi_map = lambda p, j: (0, jnp.where(p == 0, j + th, _TB - 1), 0)
    k_lo_map = lambda p, j: (jnp.where(p == 1, j, jnp.where(p == 0, 0, _NH - 1)), 0)
    k_hi_map = lambda p, j: (jnp.where(p == 1, j + _NH,
                                       jnp.where(p == 0, _NH, _NB - 1)), 0)
    v_lo_map = lambda p, j: (jnp.where(p == 2, j, 0), 0)
    v_hi_map = lambda p, j: (jnp.where(p == 2, j + _NH, _NH), 0)

    out = pl.pallas_call(
        _body,
        grid=(3, _NH),
        in_specs=[
            pl.BlockSpec((B, _BT, D), qs_lo_map),
            pl.BlockSpec((B, _BT, D), qs_hi_map),
            pl.BlockSpec((_BN, D), k_lo_map),
            pl.BlockSpec((_BN, D), k_hi_map),
            pl.BlockSpec((_BN, D), v_lo_map),
            pl.BlockSpec((_BN, D), v_hi_map),
            pl.BlockSpec((_NB, 1, _BN), lambda p, j: (0, 0, 0)),
            pl.BlockSpec((_NB, 1, _BN), lambda p, j: (0, 0, 0)),
            pl.BlockSpec((_NB, 1, _BN), lambda p, j: (0, 0, 0)),
            pl.BlockSpec((_D, _D), lambda p, j: (0, 0)),
            pl.BlockSpec((1, _D), lambda p, j: (0, 0)),
            pl.BlockSpec((_D, _D), lambda p, j: (0, 0)),
            pl.BlockSpec((1, _D), lambda p, j: (0, 0)),
        ],
        out_specs=pl.BlockSpec((B, D), lambda p, j: (0, 0)),
        out_shape=jax.ShapeDtypeStruct((B, D), jnp.float32),
        scratch_shapes=[
            pltpu.VMEM((B, D), jnp.float32),
            pltpu.VMEM((_NB, B, _BN), jnp.float32),
            pltpu.VMEM((_NB, B, _BN), jnp.float32),
            pltpu.VMEM((B, D), jnp.float32),
        ],
    )(query_states, query_states, keys, keys, values, values,
      pri, ages_f, vm, Wq, bq2, Wc, bc2)
    return out


# speedup vs baseline: 1.0746x; 1.0746x over previous
import jax
import jax.numpy as jnp
import numpy as np
from jax.experimental import pallas as pl
from jax.experimental.pallas import tpu as pltpu

_N = 16384
_D = 768
_T = 2048
_B = 4
_DECAY = 0.9
_ROUNDS = 10
_THRESH = 0.5

_NB = 8
_BN = _N // _NB
_TB = 8
_BT = _T // _TB
_SCALE = np.float32(1.0 / np.sqrt(np.float32(_D)))
_R4 = _ROUNDS * _B


def _query_body(qs_ref, wq_ref, bq_ref, q_ref, qsum):
    j = pl.program_id(0)

    @pl.when(j == 0)
    def _():
        qsum[...] = jnp.zeros_like(qsum)

    qsum[...] += jnp.sum(qs_ref[...], axis=1)

    @pl.when(j == _TB - 1)
    def _():
        q = qsum[...] * (1.0 / _T)
        q_ref[...] = jax.lax.dot_general(
            q, wq_ref[...], (((1,), (1,)), ((), ())),
            preferred_element_type=jnp.float32) + bq_ref[...]


def _main_body(q_ref, keys_ref, values_ref, pri_ref, ages_ref, vm_ref,
               wc_ref, bc_ref, out_ref, p40, s0, s1, mb):
    j = pl.program_id(0)
    f32 = jnp.float32
    bf16 = jnp.bfloat16

    @pl.when(j == 0)
    def _():
        p40[...] = jnp.zeros_like(p40)
        s0[...] = jnp.zeros_like(s0)
        s1[...] = jnp.zeros_like(s1)
        mb[...] = jnp.full_like(mb, -1e30)

    s = jax.lax.dot_general(
        q_ref[...].astype(bf16), keys_ref[...].astype(bf16),
        (((1,), (1,)), ((), ())), preferred_element_type=f32) * _SCALE

    log_decay = np.float32(np.log(_DECAY))
    eff0 = pri_ref[0] * jnp.exp(ages_ref[0] * log_decay)
    vm = vm_ref[0]
    g_rows = []
    h_rows = []
    for r in range(_ROUNDS):
        eff = eff0 * np.float32(_DECAY ** r)
        h = eff + 1e-8
        g = h * (jax.nn.sigmoid((eff - _THRESH) * 10.0) * vm)
        h_rows.append(jnp.broadcast_to(h, (_B, _BN)))
        g_rows.append(jnp.broadcast_to(g, (_B, _BN)))
    g40 = jnp.concatenate(g_rows, axis=0)
    h40 = jnp.concatenate(h_rows, axis=0)

    bm = jnp.max(s, axis=1, keepdims=True)
    mb_new = jnp.maximum(mb[...], bm)
    sc = jnp.exp(mb[...] - mb_new)
    mb[...] = mb_new
    e = jnp.exp(s - mb_new)
    e40 = jnp.concatenate([e] * _ROUNDS, axis=0)
    sc40 = jnp.concatenate([sc] * _ROUNDS, axis=0)

    num40 = e40 * g40
    den40 = e40 * h40
    s1[...] = s1[...] * sc40 + jnp.sum(num40, axis=1, keepdims=True)
    s0[...] = s0[...] * sc40 + jnp.sum(den40, axis=1, keepdims=True)
    p40[...] = p40[...] * sc40 + jax.lax.dot_general(
        num40.astype(bf16), values_ref[...].astype(bf16),
        (((1,), (0,)), ((), ())), preferred_element_type=f32)

    @pl.when(j == _NB - 1)
    def _final():
        c40 = 1.0 / (s1[...] + 1e-8 * s0[...])
        wp = p40[...] * c40
        acc = jnp.zeros((_B, _D), f32)
        for r in range(_ROUNDS):
            acc = acc + wp[r * _B:(r + 1) * _B, :]
        out_ref[...] = jax.lax.dot_general(
            acc, wc_ref[...], (((1,), (1,)), ((), ())),
            preferred_element_type=f32) * (1.0 / _ROUNDS) + bc_ref[...]


@jax.jit
def kernel(query_states, keys, values, priorities, Wq, bq, Wc, bc, ages,
           valid_mask):
    B, T, D = query_states.shape

    pri = priorities.reshape(_NB, 1, _BN)
    ages_f = ages.astype(jnp.float32).reshape(_NB, 1, _BN)
    vm = valid_mask.astype(jnp.float32).reshape(_NB, 1, _BN)
    bq2 = bq.reshape(1, D)
    bc2 = bc.reshape(1, D)

    q = pl.pallas_call(
        _query_body,
        grid=(_TB,),
        in_specs=[
            pl.BlockSpec((B, _BT, D), lambda j: (0, j, 0)),
            pl.BlockSpec((_D, _D), lambda j: (0, 0)),
            pl.BlockSpec((1, _D), lambda j: (0, 0)),
        ],
        out_specs=pl.BlockSpec((B, D), lambda j: (0, 0)),
        out_shape=jax.ShapeDtypeStruct((B, D), jnp.float32),
        scratch_shapes=[pltpu.VMEM((B, D), jnp.float32)],
    )(query_states, Wq, bq2)

    out = pl.pallas_call(
        _main_body,
        grid=(_NB,),
        in_specs=[
            pl.BlockSpec((B, _D), lambda j: (0, 0)),
            pl.BlockSpec((_BN, _D), lambda j: (j, 0)),
            pl.BlockSpec((_BN, _D), lambda j: (j, 0)),
            pl.BlockSpec((1, 1, _BN), lambda j: (j, 0, 0)),
            pl.BlockSpec((1, 1, _BN), lambda j: (j, 0, 0)),
            pl.BlockSpec((1, 1, _BN), lambda j: (j, 0, 0)),
            pl.BlockSpec((_D, _D), lambda j: (0, 0)),
            pl.BlockSpec((1, _D), lambda j: (0, 0)),
        ],
        out_specs=pl.BlockSpec((B, D), lambda j: (0, 0)),
        out_shape=jax.ShapeDtypeStruct((B, D), jnp.float32),
        scratch_shapes=[
            pltpu.VMEM((_R4, _D), jnp.float32),
            pltpu.VMEM((_R4, 1), jnp.float32),
            pltpu.VMEM((_R4, 1), jnp.float32),
            pltpu.VMEM((B, 1), jnp.float32),
        ],
    )(q, keys, values, pri, ages_f, vm, Wc, bc2)
    return out


# speedup vs baseline: 1.0757x; 1.0011x over previous
import jax
import jax.numpy as jnp
import numpy as np
from jax.experimental import pallas as pl
from jax.experimental.pallas import tpu as pltpu

_N = 16384
_D = 768
_T = 2048
_B = 4
_DECAY = 0.9
_ROUNDS = 10
_THRESH = 0.5

_NB = 8
_BN = _N // _NB
_TB = 8
_BT = _T // _TB
_SCALE = np.float32(1.0 / np.sqrt(np.float32(_D)))
_R4 = _ROUNDS * _B


def _query_body(qs_ref, wq_ref, bq_ref, q_ref, qsum):
    j = pl.program_id(0)

    @pl.when(j == 0)
    def _():
        qsum[...] = jnp.zeros_like(qsum)

    qsum[...] += jnp.sum(qs_ref[...], axis=1)

    @pl.when(j == _TB - 1)
    def _():
        q = qsum[...] * (1.0 / _T)
        q_ref[...] = jax.lax.dot_general(
            q, wq_ref[...], (((1,), (1,)), ((), ())),
            preferred_element_type=jnp.float32) + bq_ref[...]


def _main_body(q_ref, keys_ref, values_ref, pri_ref, ages_ref, vm_ref,
               wc_ref, bc_ref, out_ref, p40, s0, s1, mb):
    j = pl.program_id(0)
    f32 = jnp.float32
    bf16 = jnp.bfloat16

    @pl.when(j == 0)
    def _():
        p40[...] = jnp.zeros_like(p40)
        s0[...] = jnp.zeros_like(s0)
        s1[...] = jnp.zeros_like(s1)
        mb[...] = jnp.full_like(mb, -1e30)

    s = jax.lax.dot_general(
        q_ref[...].astype(bf16), keys_ref[...].astype(bf16),
        (((1,), (1,)), ((), ())), preferred_element_type=f32) * _SCALE

    log_decay = np.float32(np.log(_DECAY))
    eff0 = pri_ref[0] * jnp.exp(ages_ref[0] * log_decay)
    dpow = jnp.exp(log_decay * jax.lax.broadcasted_iota(
        jnp.int32, (_ROUNDS, 1), 0).astype(jnp.float32))
    eff_stack = dpow * eff0
    h_stack = eff_stack + 1e-8
    g_stack = h_stack * (jax.nn.sigmoid((eff_stack - _THRESH) * 10.0)
                         * vm_ref[0])
    g40 = jnp.repeat(g_stack, _B, axis=0)
    h40 = jnp.repeat(h_stack, _B, axis=0)

    bm = jnp.max(s, axis=1, keepdims=True)
    mb_new = jnp.maximum(mb[...], bm)
    sc = jnp.exp(mb[...] - mb_new)
    mb[...] = mb_new
    e = jnp.exp(s - mb_new)
    e40 = jnp.concatenate([e] * _ROUNDS, axis=0)
    sc40 = jnp.concatenate([sc] * _ROUNDS, axis=0)

    num40 = e40 * g40
    den40 = e40 * h40
    s1[...] = s1[...] * sc40 + jnp.sum(num40, axis=1, keepdims=True)
    s0[...] = s0[...] * sc40 + jnp.sum(den40, axis=1, keepdims=True)
    p40[...] = p40[...] * sc40 + jax.lax.dot_general(
        num40.astype(bf16), values_ref[...].astype(bf16),
        (((1,), (0,)), ((), ())), preferred_element_type=f32)

    @pl.when(j == _NB - 1)
    def _final():
        c40 = 1.0 / (s1[...] + 1e-8 * s0[...])
        wp = p40[...] * c40
        acc = jnp.zeros((_B, _D), f32)
        for r in range(_ROUNDS):
            acc = acc + wp[r * _B:(r + 1) * _B, :]
        out_ref[...] = jax.lax.dot_general(
            acc, wc_ref[...], (((1,), (1,)), ((), ())),
            preferred_element_type=f32) * (1.0 / _ROUNDS) + bc_ref[...]


@jax.jit
def kernel(query_states, keys, values, priorities, Wq, bq, Wc, bc, ages,
           valid_mask):
    B, T, D = query_states.shape

    pri = priorities.reshape(_NB, 1, _BN)
    ages_f = ages.astype(jnp.float32).reshape(_NB, 1, _BN)
    vm = valid_mask.astype(jnp.float32).reshape(_NB, 1, _BN)
    bq2 = bq.reshape(1, D)
    bc2 = bc.reshape(1, D)

    q = pl.pallas_call(
        _query_body,
        grid=(_TB,),
        in_specs=[
            pl.BlockSpec((B, _BT, D), lambda j: (0, j, 0)),
            pl.BlockSpec((_D, _D), lambda j: (0, 0)),
            pl.BlockSpec((1, _D), lambda j: (0, 0)),
        ],
        out_specs=pl.BlockSpec((B, D), lambda j: (0, 0)),
        out_shape=jax.ShapeDtypeStruct((B, D), jnp.float32),
        scratch_shapes=[pltpu.VMEM((B, D), jnp.float32)],
    )(query_states, Wq, bq2)

    out = pl.pallas_call(
        _main_body,
        grid=(_NB,),
        in_specs=[
            pl.BlockSpec((B, _D), lambda j: (0, 0)),
            pl.BlockSpec((_BN, _D), lambda j: (j, 0)),
            pl.BlockSpec((_BN, _D), lambda j: (j, 0)),
            pl.BlockSpec((1, 1, _BN), lambda j: (j, 0, 0)),
            pl.BlockSpec((1, 1, _BN), lambda j: (j, 0, 0)),
            pl.BlockSpec((1, 1, _BN), lambda j: (j, 0, 0)),
            pl.BlockSpec((_D, _D), lambda j: (0, 0)),
            pl.BlockSpec((1, _D), lambda j: (0, 0)),
        ],
        out_specs=pl.BlockSpec((B, D), lambda j: (0, 0)),
        out_shape=jax.ShapeDtypeStruct((B, D), jnp.float32),
        scratch_shapes=[
            pltpu.VMEM((_R4, _D), jnp.float32),
            pltpu.VMEM((_R4, 1), jnp.float32),
            pltpu.VMEM((_R4, 1), jnp.float32),
            pltpu.VMEM((B, 1), jnp.float32),
        ],
    )(q, keys, values, pri, ages_f, vm, Wc, bc2)
    return out
